# baseline (speedup 1.0000x reference)
import functools

import jax
import jax.numpy as jnp
from jax import lax
from jax.experimental import pallas as pl
from jax.experimental.pallas import tpu as pltpu
from jax.experimental.pallas import tpu_sc as plsc

VOCAB = 1000000
EMB = 64
B = 32
L = 700
LP = 704
NT = 16
PT = LP // NT
NC = 2
LANES = 16


_mesh = plsc.VectorSubcoreMesh(core_axis_name="c", subcore_axis_name="s")


@functools.partial(
    pl.kernel,
    out_type=jax.ShapeDtypeStruct((B,), jnp.float32),
    mesh=_mesh,
    compiler_params=pltpu.CompilerParams(
        needs_layout_passes=False, use_tc_tiling_on_sc=False),
    scratch_types=[
        pltpu.VMEM((NT, PT), jnp.int32),
        pltpu.VMEM((NT, PT, EMB), jnp.float32),
        pltpu.VMEM((PT, EMB), jnp.float32),
        pltpu.VMEM((LANES,), jnp.float32),
        pltpu.VMEM((LANES,), jnp.float32),
        pltpu.VMEM((LANES,), jnp.float32),
        pltpu.VMEM((LANES, LANES), jnp.float32),
        pltpu.VMEM_SHARED((LANES,), jnp.float32),
        pltpu.SemaphoreType.DMA,
    ],
)
def _lr_sc_kernel(xt_hbm, table_hbm, wt_hbm, b_hbm, out_hbm,
                  idx_v, rows_v, w_v, b_v, pv_v, res_v, pm_v, acc_sh, sem):
    c = lax.axis_index("c")
    s = lax.axis_index("s")

    pltpu.sync_copy(xt_hbm.at[s, pl.ds(pl.multiple_of(c * NT, NT), NT)], idx_v)

    gathers = [
        pltpu.async_copy(table_hbm.at[idx_v.at[i]], rows_v.at[i], sem)
        for i in range(NT)
    ]

    @pl.when(s == 0)
    def _():
        res_v[:] = jnp.zeros((LANES,), jnp.float32)
        pltpu.sync_copy(res_v, acc_sh)
        pltpu.sync_copy(b_hbm, b_v)

    pltpu.sync_copy(wt_hbm.at[s], w_v)

    for g in gathers:
        g.wait()

    def body(lp, accs):
        w0 = w_v[lp, pl.ds(0, 16)]
        w1 = w_v[lp, pl.ds(16, 16)]
        w2 = w_v[lp, pl.ds(32, 16)]
        w3 = w_v[lp, pl.ds(48, 16)]
        out = []
        for i in range(NT):
            a = accs[i]
            a = a + rows_v[i, lp, pl.ds(0, 16)] * w0
            a = a + rows_v[i, lp, pl.ds(16, 16)] * w1
            a = a + rows_v[i, lp, pl.ds(32, 16)] * w2
            a = a + rows_v[i, lp, pl.ds(48, 16)] * w3
            out.append(a)
        return tuple(out)

    zeros = jnp.zeros((LANES,), jnp.float32)
    accs = lax.fori_loop(0, PT, body, (zeros,) * NT)

    for i in range(NT):
        pm_v[i, :] = accs[i]
    lane = lax.iota(jnp.int32, LANES)
    partials = zeros
    for k in range(LANES):
        col = plsc.load_gather(pm_v, [lane, jnp.full((LANES,), k, jnp.int32)])
        partials = partials + col
    pv_v[:] = partials

    plsc.subcore_barrier()
    pltpu.sync_copy(pv_v, acc_sh.at[lax.iota(jnp.int32, LANES)], add=True)
    plsc.subcore_barrier()

    @pl.when(s == 0)
    def _():
        pltpu.sync_copy(acc_sh, res_v)
        z = res_v[:] + b_v[:]
        res_v[:] = 1.0 / (1.0 + jnp.exp(-z))
        pltpu.sync_copy(res_v, out_hbm.at[pl.ds(pl.multiple_of(c * NT, NT), NT)])


def kernel(x, table, W, b):
    xp = jnp.pad(x, ((0, 0), (0, LP - L)))
    xt = xp.reshape(B, NT, PT).transpose(1, 0, 2)
    wr = W.reshape(L, EMB)
    wt = jnp.pad(wr, ((0, LP - L), (0, 0))).reshape(NT, PT, EMB)
    b16 = jnp.broadcast_to(b, (LANES,)).astype(jnp.float32)
    y = _lr_sc_kernel(xt, table, wt, b16)
    return y.reshape(B, 1)
